# baseline (speedup 1.0000x reference)
import functools
import jax
import jax.numpy as jnp
from jax import lax
from jax.experimental import pallas as pl
from jax.experimental.pallas import tpu as pltpu
from jax.experimental.pallas import tpu_sc as plsc

NUM_CORES = 2
NUM_SUBCORES = 16
NUM_WORKERS = NUM_CORES * NUM_SUBCORES
LANES = 16
BLK = 128


def _make_sc_kernel(H, B, D):
    n_bt = B // BLK
    assert n_bt == NUM_WORKERS and D % 8 == 0
    DT = D // 8
    TSZ = 8 * BLK
    out_elems = H * DT * n_bt * TSZ

    mesh = plsc.VectorSubcoreMesh(core_axis_name="c", subcore_axis_name="s")

    @functools.partial(
        pl.kernel,
        out_type=jax.ShapeDtypeStruct((out_elems,), jnp.float32),
        mesh=mesh,
        scratch_types=[
            pltpu.VMEM((H, BLK), jnp.int32),
            [pltpu.VMEM((BLK, D), jnp.float32) for _ in range(2)],
            [pltpu.VMEM((DT * TSZ,), jnp.float32) for _ in range(2)],
            [pltpu.SemaphoreType.DMA for _ in range(2)],
            [pltpu.SemaphoreType.DMA for _ in range(2)],
            pltpu.SemaphoreType.DMA,
        ],
        compiler_params=pltpu.CompilerParams(
            use_tc_tiling_on_sc=False, needs_layout_passes=False
        ),
    )
    def emb_kernel(tokens_hbm, table_hbm, out_hbm, idx_v, g, t, gsems, wsems,
                   isem):
        wid = lax.axis_index("s") * NUM_CORES + lax.axis_index("c")
        pltpu.async_copy(
            tokens_hbm.at[:, pl.ds(wid * BLK, BLK)], idx_v, isem
        ).wait()

        pltpu.async_copy(table_hbm.at[idx_v.at[0]], g[0], gsems[0])

        lane_iota = lax.broadcasted_iota(jnp.int32, (LANES,), 0)
        scatter_base = [(lane_iota + k * LANES) * BLK for k in range(D // LANES)]

        def out_copies(h, b):
            return [
                pltpu.make_async_copy(
                    t[b].at[pl.ds(dt * TSZ, TSZ)],
                    out_hbm.at[pl.ds(((h * DT + dt) * n_bt + wid) * TSZ, TSZ)],
                    wsems[b],
                )
                for dt in range(DT)
            ]

        def unit(h, b):
            pltpu.make_async_copy(
                table_hbm.at[idx_v.at[h]], g[b], gsems[b]
            ).wait()

            @pl.when(h + 1 < H)
            def _():
                pltpu.async_copy(
                    table_hbm.at[idx_v.at[h + 1]], g[1 - b], gsems[1 - b]
                )

            @pl.when(h >= 2)
            def _():
                for c in out_copies(h - 2, b):
                    c.wait()

            def row_body(row, c):
                for k in range(D // LANES):
                    vals = g[b][row, pl.ds(k * LANES, LANES)]
                    plsc.store_scatter(
                        t[b], [scatter_base[k] + row], vals * 8.0
                    )
                return c

            lax.fori_loop(0, BLK, row_body, 0)

            for c in out_copies(h, b):
                c.start()

        def body(i, carry):
            unit(i * 2, 0)
            unit(i * 2 + 1, 1)
            return carry

        lax.fori_loop(0, H // 2, body, 0)

        for b in range(2):
            for c in out_copies(H - 2 + b, b):
                c.wait()

    return emb_kernel


@jax.jit
def kernel(tokens, emb_table):
    B, H = tokens.shape
    V, D = emb_table.shape
    tok_t = tokens.T.astype(jnp.int32)
    flat = _make_sc_kernel(H, B, D)(tok_t, emb_table)
    out5 = flat.reshape(H, D // 8, B // BLK, 8, BLK)
    out = jnp.transpose(out5, (2, 4, 0, 1, 3)).reshape(B, H, D)
    return out


# speedup vs baseline: 1.0146x; 1.0146x over previous
import functools
import jax
import jax.numpy as jnp
from jax import lax
from jax.experimental import pallas as pl
from jax.experimental.pallas import tpu as pltpu
from jax.experimental.pallas import tpu_sc as plsc

NUM_CORES = 2
NUM_SUBCORES = 16
NUM_WORKERS = NUM_CORES * NUM_SUBCORES
LANES = 16
BLK = 128


def _make_sc_kernel(H, B, D):
    n_bt = B // BLK
    assert n_bt == NUM_WORKERS and D % 8 == 0
    DT = D // 8
    TSZ = 8 * BLK
    out_elems = H * DT * n_bt * TSZ

    mesh = plsc.VectorSubcoreMesh(core_axis_name="c", subcore_axis_name="s")

    @functools.partial(
        pl.kernel,
        out_type=jax.ShapeDtypeStruct((out_elems,), jnp.float32),
        mesh=mesh,
        scratch_types=[
            pltpu.VMEM((H, BLK), jnp.int32),
            [pltpu.VMEM((BLK, D), jnp.float32) for _ in range(2)],
            [pltpu.VMEM((DT * TSZ,), jnp.float32) for _ in range(2)],
            [pltpu.SemaphoreType.DMA for _ in range(2)],
            [pltpu.SemaphoreType.DMA for _ in range(2)],
            pltpu.SemaphoreType.DMA,
        ],
        compiler_params=pltpu.CompilerParams(
            use_tc_tiling_on_sc=False, needs_layout_passes=False
        ),
    )
    def emb_kernel(tokens_hbm, table_hbm, out_hbm, idx_v, g, t, gsems, wsems,
                   isem):
        wid = lax.axis_index("s") * NUM_CORES + lax.axis_index("c")
        pltpu.async_copy(
            tokens_hbm.at[:, pl.ds(wid * BLK, BLK)], idx_v, isem
        ).wait()

        pltpu.async_copy(table_hbm.at[idx_v.at[0]], g[0], gsems[0])

        lane_iota = lax.broadcasted_iota(jnp.int32, (LANES,), 0)
        scatter_base = [(lane_iota + k * LANES) * BLK for k in range(D // LANES)]

        def out_copies(h, b):
            return [
                pltpu.make_async_copy(
                    t[b].at[pl.ds(dt * TSZ, TSZ)],
                    out_hbm.at[pl.ds(((h * DT + dt) * n_bt + wid) * TSZ, TSZ)],
                    wsems[b],
                )
                for dt in range(DT)
            ]

        def unit(h, b):
            pltpu.make_async_copy(
                table_hbm.at[idx_v.at[h]], g[b], gsems[b]
            ).wait()

            @pl.when(h + 1 < H)
            def _():
                pltpu.async_copy(
                    table_hbm.at[idx_v.at[h + 1]], g[1 - b], gsems[1 - b]
                )

            @pl.when(h >= 2)
            def _():
                for c in out_copies(h - 2, b):
                    c.wait()

            def row_body(row, c):
                for k in range(D // LANES):
                    vals = g[b][row, pl.ds(k * LANES, LANES)]
                    plsc.store_scatter(
                        t[b], [scatter_base[k] + row], vals * 8.0
                    )
                return c

            lax.fori_loop(0, BLK, row_body, 0, unroll=8)

            for c in out_copies(h, b):
                c.start()

        def body(i, carry):
            unit(i * 2, 0)
            unit(i * 2 + 1, 1)
            return carry

        lax.fori_loop(0, H // 2, body, 0)

        for b in range(2):
            for c in out_copies(H - 2 + b, b):
                c.wait()

    return emb_kernel


@jax.jit
def kernel(tokens, emb_table):
    B, H = tokens.shape
    V, D = emb_table.shape
    tok_t = tokens.T.astype(jnp.int32)
    flat = _make_sc_kernel(H, B, D)(tok_t, emb_table)
    out5 = flat.reshape(H, D // 8, B // BLK, 8, BLK)
    out = jnp.transpose(out5, (2, 4, 0, 1, 3)).reshape(B, H, D)
    return out


# speedup vs baseline: 1.9741x; 1.9457x over previous
import functools
import jax
import jax.numpy as jnp
from jax import lax
from jax.experimental import pallas as pl
from jax.experimental.pallas import tpu as pltpu
from jax.experimental.pallas import tpu_sc as plsc

NUM_CORES = 2
NUM_SUBCORES = 16
NUM_WORKERS = NUM_CORES * NUM_SUBCORES
LANES = 16
CHUNK = 128
NBUF = 4
AHEAD = 2
PAD = 128


def _make_sc_kernel(B, D):
    assert B % (NUM_WORKERS * CHUNK * NBUF) == 0
    b_per_w = B // NUM_WORKERS
    n_chunks = b_per_w // CHUNK

    mesh = plsc.VectorSubcoreMesh(core_axis_name="c", subcore_axis_name="s")

    @functools.partial(
        pl.kernel,
        out_type=jax.ShapeDtypeStruct((B, PAD), jnp.float32),
        mesh=mesh,
        scratch_types=[
            pltpu.VMEM((n_chunks, CHUNK), jnp.int32),
            [pltpu.VMEM((CHUNK, D), jnp.float32) for _ in range(NBUF)],
            [pltpu.SemaphoreType.DMA for _ in range(NBUF)],
            [pltpu.SemaphoreType.DMA for _ in range(NBUF)],
        ],
        compiler_params=pltpu.CompilerParams(
            use_tc_tiling_on_sc=False, needs_layout_passes=False
        ),
    )
    def emb_kernel(tokens_hbm, table_hbm, out_hbm, idx_v, rows, gsems, wsems):
        wid = lax.axis_index("s") * NUM_CORES + lax.axis_index("c")
        base = wid * b_per_w
        pltpu.sync_copy(tokens_hbm.at[pl.ds(wid * n_chunks, n_chunks)], idx_v)

        for k in range(AHEAD):
            pltpu.async_copy(table_hbm.at[idx_v.at[k]], rows[k], gsems[k])

        def scale(buf):
            def scale_row(i, c):
                for j in range(D // LANES):
                    sl = pl.ds(j * LANES, LANES)
                    buf[i, sl] = buf[i, sl] * 8.0
                return c

            lax.fori_loop(0, CHUNK, scale_row, 0, unroll=8)

        def wcopy(b, k):
            return pltpu.make_async_copy(
                rows[b],
                out_hbm.at[pl.ds(base + k * CHUNK, CHUNK), pl.ds(0, D)],
                wsems[b],
            )

        def body(g, carry):
            for b in range(NBUF):
                k = g * NBUF + b
                pltpu.make_async_copy(
                    table_hbm.at[idx_v.at[k]], rows[b], gsems[b]
                ).wait()
                scale(rows[b])
                wcopy(b, k).start()
                nb = (b + AHEAD) % NBUF
                kn = k + AHEAD

                @pl.when(kn < n_chunks)
                def _():
                    @pl.when(kn >= NBUF)
                    def _():
                        wcopy(nb, kn - NBUF).wait()

                    pltpu.async_copy(
                        table_hbm.at[idx_v.at[kn]], rows[nb], gsems[nb]
                    )

            return carry

        lax.fori_loop(0, n_chunks // NBUF, body, 0)

        for b in range(NBUF):
            wcopy(b, n_chunks - NBUF + b).wait()

    return emb_kernel


@jax.jit
def kernel(tokens, emb_table):
    B = tokens.shape[0] * tokens.shape[1]
    V, D = emb_table.shape
    padded = jnp.pad(emb_table, ((0, 0), (0, PAD - D)))
    view = padded.reshape(V * (PAD // D), D)
    flat = (tokens.reshape(B // CHUNK, CHUNK) * (PAD // D)).astype(jnp.int32)
    out_pad = _make_sc_kernel(B, D)(flat, view)
    out = out_pad[:, :D].reshape(tokens.shape + (D,))
    return out


# speedup vs baseline: 2.0065x; 1.0164x over previous
import functools
import jax
import jax.numpy as jnp
from jax import lax
from jax.experimental import pallas as pl
from jax.experimental.pallas import tpu as pltpu
from jax.experimental.pallas import tpu_sc as plsc

NUM_CORES = 2
NUM_SUBCORES = 16
NUM_WORKERS = NUM_CORES * NUM_SUBCORES
LANES = 16
CHUNK = 128
NBUF = 4
AHEAD = 2
PAD = 128


def _make_sc_kernel(B, D):
    assert B % (NUM_WORKERS * CHUNK * NBUF) == 0
    b_per_w = B // NUM_WORKERS
    n_chunks = b_per_w // CHUNK

    mesh = plsc.VectorSubcoreMesh(core_axis_name="c", subcore_axis_name="s")

    @functools.partial(
        pl.kernel,
        out_type=jax.ShapeDtypeStruct((B, PAD), jnp.float32),
        mesh=mesh,
        scratch_types=[
            pltpu.VMEM((n_chunks, CHUNK), jnp.int32),
            [pltpu.VMEM((CHUNK, D), jnp.float32) for _ in range(NBUF)],
            [pltpu.SemaphoreType.DMA for _ in range(NBUF)],
            [pltpu.SemaphoreType.DMA for _ in range(NBUF)],
        ],
        compiler_params=pltpu.CompilerParams(
            use_tc_tiling_on_sc=False, needs_layout_passes=False
        ),
    )
    def emb_kernel(tokens_hbm, table_hbm, out_hbm, idx_v, rows, gsems, wsems):
        wid = lax.axis_index("s") * NUM_CORES + lax.axis_index("c")
        base = wid * b_per_w
        pltpu.sync_copy(tokens_hbm.at[pl.ds(wid * n_chunks, n_chunks)], idx_v)

        for k in range(AHEAD):
            pltpu.async_copy(table_hbm.at[idx_v.at[k]], rows[k], gsems[k])

        def wcopy(b, k):
            return pltpu.make_async_copy(
                rows[b],
                out_hbm.at[pl.ds(base + k * CHUNK, CHUNK), pl.ds(0, D)],
                wsems[b],
            )

        def body(g, carry):
            for b in range(NBUF):
                k = g * NBUF + b
                pltpu.make_async_copy(
                    table_hbm.at[idx_v.at[k]], rows[b], gsems[b]
                ).wait()
                wcopy(b, k).start()
                nb = (b + AHEAD) % NBUF
                kn = k + AHEAD

                @pl.when(kn < n_chunks)
                def _():
                    @pl.when(kn >= NBUF)
                    def _():
                        wcopy(nb, kn - NBUF).wait()

                    pltpu.async_copy(
                        table_hbm.at[idx_v.at[kn]], rows[nb], gsems[nb]
                    )

            return carry

        lax.fori_loop(0, n_chunks // NBUF, body, 0)

        for b in range(NBUF):
            wcopy(b, n_chunks - NBUF + b).wait()

    return emb_kernel


@jax.jit
def kernel(tokens, emb_table):
    B = tokens.shape[0] * tokens.shape[1]
    V, D = emb_table.shape
    padded = jnp.pad(emb_table, ((0, 0), (0, PAD - D))) * float(D) ** 0.5
    view = padded.reshape(V * (PAD // D), D)
    flat = (tokens.reshape(B // CHUNK, CHUNK) * (PAD // D)).astype(jnp.int32)
    out_pad = _make_sc_kernel(B, D)(flat, view)
    out = out_pad[:, :D].reshape(tokens.shape + (D,))
    return out


# speedup vs baseline: 2.0237x; 1.0086x over previous
import functools
import jax
import jax.numpy as jnp
from jax import lax
from jax.experimental import pallas as pl
from jax.experimental.pallas import tpu as pltpu
from jax.experimental.pallas import tpu_sc as plsc

NUM_CORES = 2
NUM_SUBCORES = 16
NUM_WORKERS = NUM_CORES * NUM_SUBCORES
LANES = 16
CHUNK = 128
NBUF = 8
AHEAD = 4
PAD = 128


def _make_sc_kernel(B, D):
    assert B % (NUM_WORKERS * CHUNK * NBUF) == 0
    b_per_w = B // NUM_WORKERS
    n_chunks = b_per_w // CHUNK

    mesh = plsc.VectorSubcoreMesh(core_axis_name="c", subcore_axis_name="s")

    @functools.partial(
        pl.kernel,
        out_type=jax.ShapeDtypeStruct((B, PAD), jnp.float32),
        mesh=mesh,
        scratch_types=[
            pltpu.VMEM((n_chunks, CHUNK), jnp.int32),
            [pltpu.VMEM((CHUNK, D), jnp.float32) for _ in range(NBUF)],
            [pltpu.SemaphoreType.DMA for _ in range(NBUF)],
            [pltpu.SemaphoreType.DMA for _ in range(NBUF)],
        ],
        compiler_params=pltpu.CompilerParams(
            use_tc_tiling_on_sc=False, needs_layout_passes=False
        ),
    )
    def emb_kernel(tokens_hbm, table_hbm, out_hbm, idx_v, rows, gsems, wsems):
        wid = lax.axis_index("s") * NUM_CORES + lax.axis_index("c")
        base = wid * b_per_w
        pltpu.sync_copy(tokens_hbm.at[pl.ds(wid * n_chunks, n_chunks)], idx_v)

        for k in range(AHEAD):
            pltpu.async_copy(table_hbm.at[idx_v.at[k]], rows[k], gsems[k])

        def wcopy(b, k):
            return pltpu.make_async_copy(
                rows[b],
                out_hbm.at[pl.ds(base + k * CHUNK, CHUNK), pl.ds(0, D)],
                wsems[b],
            )

        def body(g, carry):
            for b in range(NBUF):
                k = g * NBUF + b
                pltpu.make_async_copy(
                    table_hbm.at[idx_v.at[k]], rows[b], gsems[b]
                ).wait()
                wcopy(b, k).start()
                nb = (b + AHEAD) % NBUF
                kn = k + AHEAD

                @pl.when(kn < n_chunks)
                def _():
                    @pl.when(kn >= NBUF)
                    def _():
                        wcopy(nb, kn - NBUF).wait()

                    pltpu.async_copy(
                        table_hbm.at[idx_v.at[kn]], rows[nb], gsems[nb]
                    )

            return carry

        lax.fori_loop(0, n_chunks // NBUF, body, 0)

        for b in range(NBUF):
            wcopy(b, n_chunks - NBUF + b).wait()

    return emb_kernel


@jax.jit
def kernel(tokens, emb_table):
    B = tokens.shape[0] * tokens.shape[1]
    V, D = emb_table.shape
    padded = jnp.pad(emb_table, ((0, 0), (0, PAD - D))) * float(D) ** 0.5
    view = padded.reshape(V * (PAD // D), D)
    flat = (tokens.reshape(B // CHUNK, CHUNK) * (PAD // D)).astype(jnp.int32)
    out_pad = _make_sc_kernel(B, D)(flat, view)
    out = out_pad[:, :D].reshape(tokens.shape + (D,))
    return out
